# Initial kernel scaffold; baseline (speedup 1.0000x reference)
import jax
import jax.numpy as jnp
from jax.experimental import pallas as pl


def kernel(seq, adj, weight_t, bias, alpha):
    raise NotImplementedError("write your pallas kernel here")


# speedup vs baseline: 2.1486x; 2.1486x over previous
import jax
import jax.numpy as jnp
from jax.experimental import pallas as pl
from jax.experimental.pallas import tpu as pltpu


def _round_up(x, m):
    return ((x + m - 1) // m) * m


def _fts_kernel(seq_ref, w_ref, out_ref):
    acc = jnp.dot(seq_ref[...].astype(jnp.bfloat16),
                  w_ref[...].astype(jnp.bfloat16),
                  preferred_element_type=jnp.float32)
    out_ref[...] = acc.astype(jnp.bfloat16)


def _agg_kernel(alpha_ref, adj_ref, fts_ref, bias_ref, out_ref):
    a16 = adj_ref[...].astype(jnp.bfloat16)
    acc = jnp.dot(a16, fts_ref[...], preferred_element_type=jnp.float32)
    out = acc + bias_ref[...]
    alpha = alpha_ref[0]
    out_ref[...] = jnp.where(out >= 0.0, out, alpha * out)


def kernel(seq, adj, weight_t, bias, alpha):
    N, in_ft = seq.shape
    out_ft = weight_t.shape[1]

    Fin = _round_up(in_ft, 128)
    Fout = _round_up(out_ft, 128)

    tm = min(512, _round_up(N, 8))
    Mp = _round_up(N, tm)
    Kp = _round_up(N, 128)

    seq_p = seq
    if (Kp, Fin) != seq.shape:
        seq_p = jnp.zeros((Kp, Fin), jnp.float32).at[:N, :in_ft].set(seq)
    w_p = weight_t
    if (Fin, Fout) != weight_t.shape:
        w_p = jnp.zeros((Fin, Fout), jnp.float32).at[:in_ft, :out_ft].set(weight_t)
    adj_p = adj
    if (Mp, Kp) != adj.shape:
        adj_p = jnp.zeros((Mp, Kp), jnp.float32).at[:N, :N].set(adj)
    bias_p = jnp.zeros((1, Fout), jnp.float32).at[0, :out_ft].set(bias)
    alpha_arr = jnp.asarray([alpha], dtype=jnp.float32)

    g1 = Kp // min(512, Kp)
    t1 = Kp // g1
    fts = pl.pallas_call(
        _fts_kernel,
        out_shape=jax.ShapeDtypeStruct((Kp, Fout), jnp.bfloat16),
        grid_spec=pltpu.PrefetchScalarGridSpec(
            num_scalar_prefetch=0,
            grid=(g1,),
            in_specs=[
                pl.BlockSpec((t1, Fin), lambda i: (i, 0)),
                pl.BlockSpec((Fin, Fout), lambda i: (0, 0)),
            ],
            out_specs=pl.BlockSpec((t1, Fout), lambda i: (i, 0)),
        ),
        compiler_params=pltpu.CompilerParams(
            dimension_semantics=("parallel",)),
        cost_estimate=pl.CostEstimate(
            flops=2 * Kp * Fin * Fout, transcendentals=0,
            bytes_accessed=4 * (Kp * Fin + Fin * Fout) + 2 * Kp * Fout),
    )(seq_p, w_p)

    grid = (Mp // tm,)
    out_p = pl.pallas_call(
        _agg_kernel,
        out_shape=jax.ShapeDtypeStruct((Mp, Fout), jnp.float32),
        grid_spec=pltpu.PrefetchScalarGridSpec(
            num_scalar_prefetch=1,
            grid=grid,
            in_specs=[
                pl.BlockSpec((tm, Kp), lambda i, a: (i, 0)),
                pl.BlockSpec((Kp, Fout), lambda i, a: (0, 0)),
                pl.BlockSpec((1, Fout), lambda i, a: (0, 0)),
            ],
            out_specs=pl.BlockSpec((tm, Fout), lambda i, a: (i, 0)),
        ),
        compiler_params=pltpu.CompilerParams(
            dimension_semantics=("parallel",)),
        cost_estimate=pl.CostEstimate(
            flops=2 * Mp * Kp * Fout, transcendentals=0,
            bytes_accessed=4 * (Mp * Kp + Mp * Fout) + 2 * Kp * Fout),
    )(alpha_arr, adj_p, fts, bias_p)

    if (Mp, Fout) != (N, out_ft):
        out_p = out_p[:N, :out_ft]
    return out_p


# speedup vs baseline: 2.6303x; 1.2242x over previous
import jax
import jax.numpy as jnp
from jax.experimental import pallas as pl
from jax.experimental.pallas import tpu as pltpu


def _round_up(x, m):
    return ((x + m - 1) // m) * m


def _gcn_kernel(alpha_ref, adj_ref, seq_ref, w_ref, bias_ref, out_ref):
    a16 = adj_ref[...].astype(jnp.bfloat16)
    s16 = seq_ref[...].astype(jnp.bfloat16)
    t = jnp.dot(a16, s16, preferred_element_type=jnp.float32)
    w16 = w_ref[...].astype(jnp.bfloat16)
    acc = jnp.dot(t.astype(jnp.bfloat16), w16,
                  preferred_element_type=jnp.float32)
    out = acc + bias_ref[...]
    alpha = alpha_ref[0]
    out_ref[...] = jnp.where(out >= 0.0, out, alpha * out)


def kernel(seq, adj, weight_t, bias, alpha):
    N, in_ft = seq.shape
    out_ft = weight_t.shape[1]

    Fin = _round_up(in_ft, 128)
    Fout = _round_up(out_ft, 128)

    tm = min(512, _round_up(N, 8))
    Mp = _round_up(N, tm)
    Kp = _round_up(N, 128)

    seq_p = seq
    if (Kp, Fin) != seq.shape:
        seq_p = jnp.zeros((Kp, Fin), jnp.float32).at[:N, :in_ft].set(seq)
    w_p = weight_t
    if (Fin, Fout) != weight_t.shape:
        w_p = jnp.zeros((Fin, Fout), jnp.float32).at[:in_ft, :out_ft].set(weight_t)
    adj_p = adj
    if (Mp, Kp) != adj.shape:
        adj_p = jnp.zeros((Mp, Kp), jnp.float32).at[:N, :N].set(adj)
    bias_p = jnp.zeros((1, Fout), jnp.float32).at[0, :out_ft].set(bias)
    alpha_arr = jnp.asarray([alpha], dtype=jnp.float32)

    grid = (Mp // tm,)
    out_p = pl.pallas_call(
        _gcn_kernel,
        out_shape=jax.ShapeDtypeStruct((Mp, Fout), jnp.float32),
        grid_spec=pltpu.PrefetchScalarGridSpec(
            num_scalar_prefetch=1,
            grid=grid,
            in_specs=[
                pl.BlockSpec((tm, Kp), lambda i, a: (i, 0)),
                pl.BlockSpec((Kp, Fin), lambda i, a: (0, 0)),
                pl.BlockSpec((Fin, Fout), lambda i, a: (0, 0)),
                pl.BlockSpec((1, Fout), lambda i, a: (0, 0)),
            ],
            out_specs=pl.BlockSpec((tm, Fout), lambda i, a: (i, 0)),
        ),
        compiler_params=pltpu.CompilerParams(
            dimension_semantics=("parallel",)),
        cost_estimate=pl.CostEstimate(
            flops=2 * Mp * Kp * Fin + 2 * Mp * Fin * Fout, transcendentals=0,
            bytes_accessed=4 * (Mp * Kp + Kp * Fin + Fin * Fout + Mp * Fout)),
    )(alpha_arr, adj_p, seq_p, w_p, bias_p)

    if (Mp, Fout) != (N, out_ft):
        out_p = out_p[:N, :out_ft]
    return out_p
